# baseline (speedup 1.0000x reference)
import functools

import jax
import jax.numpy as jnp
import numpy as np
from jax import lax
from jax.experimental import pallas as pl
from jax.experimental.pallas import tpu as pltpu

WS = 16
K = 4
STEND = 8
B, T, C, NQ, Q = 16, 128, 512, 20, 300
NW = T - WS + 1
NT = T - K + 1
BB = 8
G_STEPS = B // BB

_PREC = lax.Precision.DEFAULT
_GPREC = lax.Precision.HIGHEST


def _pos_enc_table(L, D):
    pos = np.arange(L, dtype=np.float32)[:, None]
    i = np.arange(D // 2, dtype=np.float32)[None, :]
    angles = pos / np.power(10000.0, (2.0 * i) / D)
    pe = np.zeros((L, D), dtype=np.float32)
    pe[:, 0::2] = np.sin(angles)
    pe[:, 1::2] = np.cos(angles)
    return pe


_PE_VIS = _pos_enc_table(T, C)
_PE_Q = _pos_enc_table(NQ, Q)
_CONST_CACHE = {}


def _gumbel_t():
    if "g" not in _CONST_CACHE:
        try:
            with jax.ensure_compile_time_eval():
                g = jax.random.gumbel(jax.random.key(1234), (B, T, NQ),
                                      jnp.float32)
            _CONST_CACHE["g"] = np.transpose(np.asarray(g), (0, 2, 1)).copy()
        except Exception:
            g = jax.random.gumbel(jax.random.key(1234), (B, T, NQ),
                                  jnp.float32)
            return jnp.transpose(g, (0, 2, 1))
    return _CONST_CACHE["g"]


def _body(x_ref, q_ref, g_ref, vid_ref, pev_ref, peq_ref,
          wv1_ref, bv1_ref, ws1_ref, bs1_ref, ws2_ref, bs2_ref,
          wv2_ref, bv2_ref, wp_ref, bp_ref,
          wst_ref, bst_ref, wen_ref, ben_ref,
          p_out, st_out, en_out, bm_scr):
    step = pl.program_id(0)
    xp = (x_ref[...] + pev_ref[...][None]).reshape(BB * T, C)

    v = lax.dot_general(xp, wv1_ref[...], (((1,), (0,)), ((), ())),
                        precision=_PREC, preferred_element_type=jnp.float32)
    v = v + bv1_ref[...]

    s2 = v + jnp.roll(v, -1, axis=0)
    u = (s2 + jnp.roll(s2, -2, axis=0)) * 0.25
    u2 = lax.dot_general(u, wv2_ref[...], (((1,), (0,)), ((), ())),
                         precision=_PREC, preferred_element_type=jnp.float32)
    u2 = u2 + bv2_ref[...]

    iota_q = lax.broadcasted_iota(jnp.int32, (NQ, T), 0)
    trow = lax.broadcasted_iota(jnp.int32, (1, T), 1)
    ms = []
    for j in range(BB):
        qp = q_ref[j] + peq_ref[...]
        enc1 = lax.dot_general(qp, ws1_ref[...], (((1,), (0,)), ((), ())),
                               precision=_PREC, preferred_element_type=jnp.float32)
        enc1 = enc1 + bs1_ref[...]
        enc2 = lax.dot_general(qp, ws2_ref[...], (((1,), (0,)), ((), ())),
                               precision=_PREC, preferred_element_type=jnp.float32)
        enc2 = enc2 + bs2_ref[...]
        vb = v[j * T:(j + 1) * T]

        simt = lax.dot_general(enc1, vb, (((1,), (1,)), ((), ())),
                               precision=_PREC, preferred_element_type=jnp.float32)
        mn = jnp.min(simt, axis=0, keepdims=True)
        p = simt - mn
        p = p / jnp.sum(p, axis=0, keepdims=True)
        lg = jnp.log(p + 1e-12) + g_ref[j]
        mx = jnp.max(lg, axis=0, keepdims=True)
        labels = jnp.min(jnp.where(lg >= mx, iota_q, NQ + 7), axis=0,
                         keepdims=True)

        l = [labels] + [jnp.roll(labels, -d, axis=1) for d in range(1, 4)]
        key = None
        for i in range(4):
            cnt = ((l[i] == l[0]).astype(jnp.int32)
                   + (l[i] == l[1]).astype(jnp.int32)
                   + (l[i] == l[2]).astype(jnp.int32)
                   + (l[i] == l[3]).astype(jnp.int32))
            ki = cnt * 64 + (63 - l[i])
            key = ki if key is None else jnp.maximum(key, ki)
        mode4 = 63 - jnp.remainder(key, 64)

        oht = (iota_q == mode4).astype(jnp.float32)
        gathered = lax.dot_general(oht, enc2, (((0,), (0,)), ((), ())),
                                   precision=_GPREC,
                                   preferred_element_type=jnp.float32)
        m = gathered * u2[j * T:(j + 1) * T]
        ms.append(m)

        vid = vid_ref[j, 0, 0]
        lb = vid * K
        bias_rows = []
        for i in range(STEND):
            s_i = (i * lb) // STEND
            e_i = ((i + 1) * lb + (STEND - 1)) // STEND
            mem = None
            for k in range(K):
                w = trow - 4 * k
                jj = 4 * trow - 15 * k
                c = (w >= 0) & (w <= NW - 1) & (jj >= s_i) & (jj < e_i)
                mem = c if mem is None else (mem | c)
            bias_rows.append(jnp.where(mem, 0.0, -jnp.inf))
        bias_t = jnp.transpose(jnp.concatenate(bias_rows, axis=0), (1, 0))
        bms = [jnp.max(m + bias_t[:, i:i + 1], axis=0, keepdims=True)
               for i in range(STEND)]
        bm_scr[pl.ds(step * BB + j, 1)] = jnp.concatenate(bms, axis=0)[None]

    p_out[...] = (lax.dot_general(jnp.concatenate(ms, axis=0), wp_ref[...],
                                  (((1,), (0,)), ((), ())), precision=_PREC,
                                  preferred_element_type=jnp.float32)
                  + bp_ref[...]).reshape(BB, T, 2)

    @pl.when(step == G_STEPS - 1)
    def _finish():
        st = bst_ref[...]
        en = ben_ref[...]
        for i in range(STEND):
            blk = bm_scr[:, i, :]
            st = st + lax.dot_general(blk, wst_ref[i], (((1,), (0,)), ((), ())),
                                      precision=_PREC,
                                      preferred_element_type=jnp.float32)
            en = en + lax.dot_general(blk, wen_ref[i], (((1,), (0,)), ((), ())),
                                      precision=_PREC,
                                      preferred_element_type=jnp.float32)
        st_out[...] = st
        en_out[...] = en


def kernel(vis_feats, query_feats, vid_lens, W_v1, b_v1, W_v2, b_v2,
           W_s1, b_s1, W_s2, b_s2, W_p, b_p, W_st, b_st, W_en, b_en):
    vid = vid_lens.astype(jnp.int32).reshape(B, 1, 1)
    wst_r = jnp.transpose(W_st.reshape(C, STEND, STEND), (1, 0, 2))
    wen_r = jnp.transpose(W_en.reshape(C, STEND, STEND), (1, 0, 2))

    rep = lambda i: (0,) * 2
    rep3 = lambda i: (0,) * 3
    p_full, st, en = pl.pallas_call(
        _body,
        grid=(G_STEPS,),
        in_specs=[
            pl.BlockSpec((BB, T, C), lambda i: (i, 0, 0)),
            pl.BlockSpec((BB, NQ, Q), lambda i: (i, 0, 0)),
            pl.BlockSpec((BB, NQ, T), lambda i: (i, 0, 0)),
            pl.BlockSpec((BB, 1, 1), lambda i: (i, 0, 0), memory_space=pltpu.SMEM),
            pl.BlockSpec((T, C), rep),
            pl.BlockSpec((NQ, Q), rep),
            pl.BlockSpec((C, C), rep),
            pl.BlockSpec((1, C), rep),
            pl.BlockSpec((Q, C), rep),
            pl.BlockSpec((1, C), rep),
            pl.BlockSpec((Q, C), rep),
            pl.BlockSpec((1, C), rep),
            pl.BlockSpec((C, C), rep),
            pl.BlockSpec((1, C), rep),
            pl.BlockSpec((C, 2), rep),
            pl.BlockSpec((1, 2), rep),
            pl.BlockSpec((STEND, C, STEND), rep3),
            pl.BlockSpec((1, STEND), rep),
            pl.BlockSpec((STEND, C, STEND), rep3),
            pl.BlockSpec((1, STEND), rep),
        ],
        out_specs=[
            pl.BlockSpec((BB, T, 2), lambda i: (i, 0, 0)),
            pl.BlockSpec((B, STEND), rep),
            pl.BlockSpec((B, STEND), rep),
        ],
        out_shape=[
            jax.ShapeDtypeStruct((B, T, 2), jnp.float32),
            jax.ShapeDtypeStruct((B, STEND), jnp.float32),
            jax.ShapeDtypeStruct((B, STEND), jnp.float32),
        ],
        scratch_shapes=[pltpu.VMEM((B, STEND, C), jnp.float32)],
    )(vis_feats, query_feats, jnp.asarray(_gumbel_t()), vid,
      jnp.asarray(_PE_VIS), jnp.asarray(_PE_Q),
      W_v1, b_v1.reshape(1, C), W_s1, b_s1.reshape(1, C),
      W_s2, b_s2.reshape(1, C), W_v2, b_v2.reshape(1, C), W_p,
      b_p.reshape(1, 2), wst_r, b_st.reshape(1, STEND), wen_r,
      b_en.reshape(1, STEND))

    out0 = jnp.stack([p_full[:, 4 * k:4 * k + NW, :] for k in range(K)],
                     axis=2)
    out0 = jnp.transpose(out0, (0, 3, 1, 2))
    return out0, st.reshape(B, STEND, 1), en.reshape(B, STEND, 1)


# speedup vs baseline: 1.1201x; 1.1201x over previous
import functools

import jax
import jax.numpy as jnp
import numpy as np
from jax import lax
from jax.experimental import pallas as pl
from jax.experimental.pallas import tpu as pltpu

WS = 16
K = 4
STEND = 8
B, T, C, NQ, Q = 16, 128, 512, 20, 300
NW = T - WS + 1
NT = T - K + 1
BB = 4
G_STEPS = B // BB

_PREC = lax.Precision.DEFAULT
_GPREC = lax.Precision.HIGHEST


def _pos_enc_table(L, D):
    pos = np.arange(L, dtype=np.float32)[:, None]
    i = np.arange(D // 2, dtype=np.float32)[None, :]
    angles = pos / np.power(10000.0, (2.0 * i) / D)
    pe = np.zeros((L, D), dtype=np.float32)
    pe[:, 0::2] = np.sin(angles)
    pe[:, 1::2] = np.cos(angles)
    return pe


_PE_VIS = _pos_enc_table(T, C)
_PE_Q = _pos_enc_table(NQ, Q)
_CONST_CACHE = {}


def _gumbel_t():
    if "g" not in _CONST_CACHE:
        try:
            with jax.ensure_compile_time_eval():
                g = jax.random.gumbel(jax.random.key(1234), (B, T, NQ),
                                      jnp.float32)
            _CONST_CACHE["g"] = np.transpose(np.asarray(g), (0, 2, 1)).copy()
        except Exception:
            g = jax.random.gumbel(jax.random.key(1234), (B, T, NQ),
                                  jnp.float32)
            return jnp.transpose(g, (0, 2, 1))
    return _CONST_CACHE["g"]


def _body(x_ref, q_ref, g_ref, vid_ref, pev_ref, peq_ref,
          wv1_ref, bv1_ref, ws1_ref, bs1_ref, ws2_ref, bs2_ref,
          wv2_ref, bv2_ref, wp_ref, bp_ref,
          wst_ref, bst_ref, wen_ref, ben_ref,
          p_out, st_out, en_out, bm_scr):
    step = pl.program_id(0)
    xp = (x_ref[...] + pev_ref[...][None]).reshape(BB * T, C)

    v = lax.dot_general(xp, wv1_ref[...], (((1,), (0,)), ((), ())),
                        precision=_PREC, preferred_element_type=jnp.float32)
    v = v + bv1_ref[...]

    s2 = v + jnp.roll(v, -1, axis=0)
    u = (s2 + jnp.roll(s2, -2, axis=0)) * 0.25
    u2 = lax.dot_general(u, wv2_ref[...], (((1,), (0,)), ((), ())),
                         precision=_PREC, preferred_element_type=jnp.float32)
    u2 = u2 + bv2_ref[...]

    iota_q = lax.broadcasted_iota(jnp.int32, (NQ, T), 0)
    trow = lax.broadcasted_iota(jnp.int32, (1, T), 1)

    enc2s, simts = [], []
    for j in range(BB):
        qp = q_ref[j] + peq_ref[...]
        enc1 = lax.dot_general(qp, ws1_ref[...], (((1,), (0,)), ((), ())),
                               precision=_PREC, preferred_element_type=jnp.float32)
        enc1 = enc1 + bs1_ref[...]
        enc2 = lax.dot_general(qp, ws2_ref[...], (((1,), (0,)), ((), ())),
                               precision=_PREC, preferred_element_type=jnp.float32)
        enc2s.append(enc2 + bs2_ref[...])
        simts.append(lax.dot_general(enc1, v[j * T:(j + 1) * T],
                                     (((1,), (1,)), ((), ())), precision=_PREC,
                                     preferred_element_type=jnp.float32))

    mode4s = []
    for j in range(BB):
        simt = simts[j]
        mn = jnp.min(simt, axis=0, keepdims=True)
        p = simt - mn
        p = p / jnp.sum(p, axis=0, keepdims=True)
        lg = jnp.log(p + 1e-12) + g_ref[j]
        mx = jnp.max(lg, axis=0, keepdims=True)
        labels = jnp.min(jnp.where(lg >= mx, iota_q, NQ + 7), axis=0,
                         keepdims=True)
        l = [labels] + [jnp.roll(labels, -d, axis=1) for d in range(1, 4)]
        key = None
        for i in range(4):
            cnt = ((l[i] == l[0]).astype(jnp.int32)
                   + (l[i] == l[1]).astype(jnp.int32)
                   + (l[i] == l[2]).astype(jnp.int32)
                   + (l[i] == l[3]).astype(jnp.int32))
            ki = cnt * 64 + (63 - l[i])
            key = ki if key is None else jnp.maximum(key, ki)
        mode4s.append(63 - jnp.remainder(key, 64))

    ms = []
    for j in range(BB):
        oht = (iota_q == mode4s[j]).astype(jnp.float32)
        gathered = lax.dot_general(oht, enc2s[j], (((0,), (0,)), ((), ())),
                                   precision=_GPREC,
                                   preferred_element_type=jnp.float32)
        ms.append(gathered * u2[j * T:(j + 1) * T])

    for j in range(BB):
        vid = vid_ref[j, 0, 0]
        lb = vid * K
        bias_rows = []
        for i in range(STEND):
            s_i = (i * lb) // STEND
            e_i = ((i + 1) * lb + (STEND - 1)) // STEND
            mem = None
            for k in range(K):
                w = trow - 4 * k
                jj = 4 * trow - 15 * k
                c = (w >= 0) & (w <= NW - 1) & (jj >= s_i) & (jj < e_i)
                mem = c if mem is None else (mem | c)
            bias_rows.append(jnp.where(mem, 0.0, -jnp.inf))
        bias_t = jnp.transpose(jnp.concatenate(bias_rows, axis=0), (1, 0))
        bms = [jnp.max(ms[j] + bias_t[:, i:i + 1], axis=0, keepdims=True)
               for i in range(STEND)]
        bm_scr[pl.ds(step * BB + j, 1)] = jnp.concatenate(bms, axis=0)[None]

    p_out[...] = (lax.dot_general(jnp.concatenate(ms, axis=0), wp_ref[...],
                                  (((1,), (0,)), ((), ())), precision=_PREC,
                                  preferred_element_type=jnp.float32)
                  + bp_ref[...]).reshape(BB, T, 2)

    @pl.when(step == G_STEPS - 1)
    def _finish():
        st = bst_ref[...]
        en = ben_ref[...]
        for i in range(STEND):
            blk = bm_scr[:, i, :]
            st = st + lax.dot_general(blk, wst_ref[i], (((1,), (0,)), ((), ())),
                                      precision=_PREC,
                                      preferred_element_type=jnp.float32)
            en = en + lax.dot_general(blk, wen_ref[i], (((1,), (0,)), ((), ())),
                                      precision=_PREC,
                                      preferred_element_type=jnp.float32)
        st_out[...] = st
        en_out[...] = en


def kernel(vis_feats, query_feats, vid_lens, W_v1, b_v1, W_v2, b_v2,
           W_s1, b_s1, W_s2, b_s2, W_p, b_p, W_st, b_st, W_en, b_en):
    vid = vid_lens.astype(jnp.int32).reshape(B, 1, 1)
    wst_r = jnp.transpose(W_st.reshape(C, STEND, STEND), (1, 0, 2))
    wen_r = jnp.transpose(W_en.reshape(C, STEND, STEND), (1, 0, 2))

    rep = lambda i: (0,) * 2
    rep3 = lambda i: (0,) * 3
    p_full, st, en = pl.pallas_call(
        _body,
        grid=(G_STEPS,),
        in_specs=[
            pl.BlockSpec((BB, T, C), lambda i: (i, 0, 0)),
            pl.BlockSpec((BB, NQ, Q), lambda i: (i, 0, 0)),
            pl.BlockSpec((BB, NQ, T), lambda i: (i, 0, 0)),
            pl.BlockSpec((BB, 1, 1), lambda i: (i, 0, 0), memory_space=pltpu.SMEM),
            pl.BlockSpec((T, C), rep),
            pl.BlockSpec((NQ, Q), rep),
            pl.BlockSpec((C, C), rep),
            pl.BlockSpec((1, C), rep),
            pl.BlockSpec((Q, C), rep),
            pl.BlockSpec((1, C), rep),
            pl.BlockSpec((Q, C), rep),
            pl.BlockSpec((1, C), rep),
            pl.BlockSpec((C, C), rep),
            pl.BlockSpec((1, C), rep),
            pl.BlockSpec((C, 2), rep),
            pl.BlockSpec((1, 2), rep),
            pl.BlockSpec((STEND, C, STEND), rep3),
            pl.BlockSpec((1, STEND), rep),
            pl.BlockSpec((STEND, C, STEND), rep3),
            pl.BlockSpec((1, STEND), rep),
        ],
        out_specs=[
            pl.BlockSpec((BB, T, 2), lambda i: (i, 0, 0)),
            pl.BlockSpec((B, STEND), rep),
            pl.BlockSpec((B, STEND), rep),
        ],
        out_shape=[
            jax.ShapeDtypeStruct((B, T, 2), jnp.float32),
            jax.ShapeDtypeStruct((B, STEND), jnp.float32),
            jax.ShapeDtypeStruct((B, STEND), jnp.float32),
        ],
        scratch_shapes=[pltpu.VMEM((B, STEND, C), jnp.float32)],
    )(vis_feats, query_feats, jnp.asarray(_gumbel_t()), vid,
      jnp.asarray(_PE_VIS), jnp.asarray(_PE_Q),
      W_v1, b_v1.reshape(1, C), W_s1, b_s1.reshape(1, C),
      W_s2, b_s2.reshape(1, C), W_v2, b_v2.reshape(1, C), W_p,
      b_p.reshape(1, 2), wst_r, b_st.reshape(1, STEND), wen_r,
      b_en.reshape(1, STEND))

    out0 = jnp.stack([p_full[:, 4 * k:4 * k + NW, :] for k in range(K)],
                     axis=2)
    out0 = jnp.transpose(out0, (0, 3, 1, 2))
    return out0, st.reshape(B, STEND, 1), en.reshape(B, STEND, 1)
